# baseline (speedup 1.0000x reference)
import jax
import jax.numpy as jnp
from jax import lax
from jax.experimental import pallas as pl
from jax.experimental.pallas import tpu as pltpu
from jax.experimental.pallas import tpu_sc as plsc

VOCAB = 100000
EMBED = 64
NUM_POS = 200
BATCH = 4096
SEQ = 200

NC = 2
NS = 16
NW = NC * NS

BW = BATCH // NW
DT = EMBED // 8
BT = BATCH // 128


def _body(idst_hbm, pos_hbm, table_hbm, out_hbm,
          idst_v, pos_v, rowsb, trb, gsems, ssems):
    wid = lax.axis_index("s") * NC + lax.axis_index("c")
    b0 = wid * BW

    pltpu.sync_copy(idst_hbm.at[:, pl.ds(b0, BW)], idst_v)
    pltpu.sync_copy(pos_hbm, pos_v)

    iota = lax.iota(jnp.int32, 16)
    dt_vecs = [(dg * 16 + iota) >> 3 for dg in range(4)]
    di_vecs = [(dg * 16 + iota) & 7 for dg in range(4)]

    def start_gather(s, rb):
        return pltpu.async_copy(
            table_hbm.at[idst_v.at[s]], rowsb[rb], gsems[rb])

    def wait_gather(s, rb):
        pltpu.make_async_copy(
            table_hbm.at[idst_v.at[s]], rowsb[rb], gsems[rb]).wait()

    def store(s, tb):
        return pltpu.make_async_copy(
            trb[tb].at[:, :, pl.ds(0, 128)], out_hbm.at[s, :, wid], ssems[tb])

    def transpose_add(s, rb, tb):
        rows = rowsb[rb]
        tr = trb[tb]
        p = [pos_v[s, pl.ds(dg * 16, 16)] for dg in range(4)]

        def jstep(j, _):
            jv = jnp.full((16,), j, jnp.int32)
            for dg in range(4):
                v = rows[j, pl.ds(dg * 16, 16)] + p[dg]
                plsc.store_scatter(tr, [dt_vecs[dg], di_vecs[dg], jv], v)
            return ()

        lax.fori_loop(0, BW, jstep, (), unroll=4)

    def process(s, rb, tb, do_wait_s, do_gather):
        wait_gather(s, rb)
        if do_gather:
            start_gather(s + 1, 1 - rb)
        if do_wait_s:
            store(s - 2, tb).wait()
        transpose_add(s, rb, tb)
        store(s, tb).start()

    start_gather(0, 0)
    process(0, 0, 0, False, True)
    process(1, 1, 1, False, True)

    def pair(t, _):
        s = 2 * t + 2
        process(s, 0,0, True, True)
        process(s + 1, 1, 1, True, True)
        return ()

    lax.fori_loop(0, (SEQ - 4) // 2, pair, ())

    process(SEQ - 2, 0, 0, True, True)
    process(SEQ - 1, 1, 1, True, False)
    store(SEQ - 2, 0).wait()
    store(SEQ - 1, 1).wait()


@jax.jit
def _run(idst, token_table, pos_table):
    kern = pl.kernel(
        _body,
        out_type=jax.ShapeDtypeStruct((SEQ, DT, BT, 8, 128), jnp.float32),
        mesh=plsc.VectorSubcoreMesh(core_axis_name="c", subcore_axis_name="s"),
        scratch_types=[
            pltpu.VMEM((SEQ, BW), jnp.int32),
            pltpu.VMEM((NUM_POS, EMBED), jnp.float32),
            [pltpu.VMEM((BW, EMBED), jnp.float32) for _ in range(2)],
            [pltpu.VMEM((DT, 8, 129), jnp.float32) for _ in range(2)],
            [pltpu.SemaphoreType.DMA for _ in range(2)],
            [pltpu.SemaphoreType.DMA for _ in range(2)],
        ],
        compiler_params=pltpu.CompilerParams(
            use_tc_tiling_on_sc=False, needs_layout_passes=False),
    )
    o = kern(idst, pos_table, token_table)
    return o.transpose((2, 4, 0, 1, 3)).reshape(BATCH, SEQ, EMBED)


def kernel(input_ids, token_table, pos_table):
    return _run(input_ids.astype(jnp.int32).T, token_table, pos_table)


# speedup vs baseline: 1.0054x; 1.0054x over previous
import jax
import jax.numpy as jnp
from jax import lax
from jax.experimental import pallas as pl
from jax.experimental.pallas import tpu as pltpu
from jax.experimental.pallas import tpu_sc as plsc

VOCAB = 100000
EMBED = 64
NUM_POS = 200
BATCH = 4096
SEQ = 200

NC = 2
NS = 16
NW = NC * NS

BW = BATCH // NW
DT = EMBED // 8
BT = BATCH // 128


def _body(idst_hbm, pos_hbm, table_hbm, out_hbm,
          idst_v, pos_v, rowsb, trb, gsems, ssems):
    wid = lax.axis_index("s") * NC + lax.axis_index("c")
    b0 = wid * BW

    pltpu.sync_copy(idst_hbm.at[:, pl.ds(b0, BW)], idst_v)
    pltpu.sync_copy(pos_hbm, pos_v)

    iota = lax.iota(jnp.int32, 16)
    dt_vecs = [(dg * 16 + iota) >> 3 for dg in range(4)]
    di_vecs = [(dg * 16 + iota) & 7 for dg in range(4)]

    def start_gather(s, rb):
        return pltpu.async_copy(
            table_hbm.at[idst_v.at[s]], rowsb[rb], gsems[rb])

    def wait_gather(s, rb):
        pltpu.make_async_copy(
            table_hbm.at[idst_v.at[s]], rowsb[rb], gsems[rb]).wait()

    def store(s, tb):
        return pltpu.make_async_copy(
            trb[tb].at[:, :, pl.ds(0, 128)], out_hbm.at[s, :, wid], ssems[tb])

    def transpose_add(s, rb, tb):
        rows = rowsb[rb]
        tr = trb[tb]
        p = [pos_v[s, pl.ds(dg * 16, 16)] for dg in range(4)]

        def jstep(j, _):
            jv = jnp.full((16,), j, jnp.int32)
            for dg in range(4):
                v = rows[j, pl.ds(dg * 16, 16)] + p[dg]
                plsc.store_scatter(tr, [dt_vecs[dg], di_vecs[dg], jv], v)
            return ()

        lax.fori_loop(0, BW, jstep, (), unroll=8)

    def process(s, rb, tb, do_wait_s, do_gather):
        wait_gather(s, rb)
        if do_gather:
            start_gather(s + 1, 1 - rb)
        if do_wait_s:
            store(s - 2, tb).wait()
        transpose_add(s, rb, tb)
        store(s, tb).start()

    start_gather(0, 0)
    process(0, 0, 0, False, True)
    process(1, 1, 1, False, True)

    def pair(t, _):
        s = 2 * t + 2
        process(s, 0,0, True, True)
        process(s + 1, 1, 1, True, True)
        return ()

    lax.fori_loop(0, (SEQ - 4) // 2, pair, ())

    process(SEQ - 2, 0, 0, True, True)
    process(SEQ - 1, 1, 1, True, False)
    store(SEQ - 2, 0).wait()
    store(SEQ - 1, 1).wait()


@jax.jit
def _run(idst, token_table, pos_table):
    kern = pl.kernel(
        _body,
        out_type=jax.ShapeDtypeStruct((SEQ, DT, BT, 8, 128), jnp.float32),
        mesh=plsc.VectorSubcoreMesh(core_axis_name="c", subcore_axis_name="s"),
        scratch_types=[
            pltpu.VMEM((SEQ, BW), jnp.int32),
            pltpu.VMEM((NUM_POS, EMBED), jnp.float32),
            [pltpu.VMEM((BW, EMBED), jnp.float32) for _ in range(2)],
            [pltpu.VMEM((DT, 8, 129), jnp.float32) for _ in range(2)],
            [pltpu.SemaphoreType.DMA for _ in range(2)],
            [pltpu.SemaphoreType.DMA for _ in range(2)],
        ],
        compiler_params=pltpu.CompilerParams(
            use_tc_tiling_on_sc=False, needs_layout_passes=False),
    )
    o = kern(idst, pos_table, token_table)
    return o.transpose((2, 4, 0, 1, 3)).reshape(BATCH, SEQ, EMBED)


def kernel(input_ids, token_table, pos_table):
    return _run(input_ids.astype(jnp.int32).T, token_table, pos_table)
